# baseline (speedup 1.0000x reference)
import functools

import jax
import jax.numpy as jnp
from jax import lax
from jax.experimental import pallas as pl
from jax.experimental.pallas import tpu as pltpu
from jax.experimental.pallas import tpu_sc as plsc

EPS = 1e-12
B = 1024
V = 100000
S = 200
L = 16

_NC, _NS = 2, 16
_NW = _NC * _NS
_BPW = B // _NW


def _sc_gather_select(pv_flat, ad_flat, pg, tidx, cidx):
    mesh = plsc.VectorSubcoreMesh(core_axis_name="c", subcore_axis_name="s")

    @functools.partial(
        pl.kernel,
        mesh=mesh,
        out_type=jax.ShapeDtypeStruct((B,), jnp.float32),
        scratch_types=[
            pltpu.VMEM((_BPW,), jnp.int32),
            pltpu.VMEM((_BPW,), jnp.int32),
            pltpu.VMEM((_BPW,), jnp.float32),
            pltpu.VMEM((_BPW,), jnp.int32),
            pltpu.VMEM((_BPW,), jnp.int32),
            pltpu.VMEM((_BPW,), jnp.float32),
            pltpu.VMEM((_BPW,), jnp.float32),
            pltpu.VMEM((_BPW,), jnp.float32),
            pltpu.SemaphoreType.DMA,
            pltpu.SemaphoreType.DMA,
        ],
    )
    def k(pv_hbm, ad_hbm, pg_hbm, tidx_hbm, cidx_hbm, out_hbm,
          tidx_v, cidx_v, pg_v, vidx_v, aidx_v, pv_v, pc_v, a_v,
          sem0, sem1):
        wid = lax.axis_index("s") * _NC + lax.axis_index("c")
        base = wid * _BPW

        pltpu.sync_copy(tidx_hbm.at[pl.ds(base, _BPW)], tidx_v)
        pltpu.sync_copy(cidx_hbm.at[pl.ds(base, _BPW)], cidx_v)
        pltpu.sync_copy(pg_hbm.at[pl.ds(base, _BPW)], pg_v)

        for j in range(_BPW // L):
            row = base + j * L + lax.iota(jnp.int32, L)
            tv = tidx_v[pl.ds(j * L, L)]
            cv = cidx_v[pl.ds(j * L, L)]
            tv_safe = jnp.minimum(jnp.maximum(tv, 0), V - 1)
            vidx_v[pl.ds(j * L, L)] = row * V + tv_safe
            aidx_v[pl.ds(j * L, L)] = row * S + cv

        cp0 = pltpu.async_copy(pv_hbm.at[vidx_v], pv_v, sem0)
        cp1 = pltpu.async_copy(ad_hbm.at[aidx_v], pc_v, sem1)
        cp0.wait()
        cp1.wait()

        for j in range(_BPW // L):
            sl = pl.ds(j * L, L)
            tv = tidx_v[sl]
            pgv = pg_v[sl]
            gen = pgv * pv_v[sl] + EPS
            cpy = (1.0 - pgv) * pc_v[sl] + EPS
            a_v[sl] = jnp.where(tv < V, gen, cpy)

        pltpu.sync_copy(a_v, out_hbm.at[pl.ds(base, _BPW)])

    return k(pv_flat, ad_flat, pg, tidx, cidx)


def _tc_neg_mean_log(a):
    def body(a_ref, o_ref):
        x = a_ref[...]
        o_ref[0, 0] = -jnp.sum(jnp.log(x)) * (1.0 / B)

    out = pl.pallas_call(
        body,
        out_shape=jax.ShapeDtypeStruct((1, 1), jnp.float32),
        in_specs=[pl.BlockSpec(memory_space=pltpu.VMEM)],
        out_specs=pl.BlockSpec(memory_space=pltpu.SMEM),
    )(a.reshape(8, B // 8))
    return out[0, 0]


def kernel(P_vocab, attn_dist, p_gen, target_idx, copy_idx):
    pv_flat = P_vocab.reshape(B * V)
    ad_flat = attn_dist.reshape(B * S)
    pg = p_gen.reshape(B)
    tidx = target_idx.astype(jnp.int32)
    cidx = copy_idx.astype(jnp.int32)
    a = _sc_gather_select(pv_flat, ad_flat, pg, tidx, cidx)
    return _tc_neg_mean_log(a)


# speedup vs baseline: 2.3377x; 2.3377x over previous
import functools

import jax
import jax.numpy as jnp
from jax import lax
from jax.experimental import pallas as pl
from jax.experimental.pallas import tpu as pltpu
from jax.experimental.pallas import tpu_sc as plsc

EPS = 1e-12
B = 1024
V = 100000
S = 200
L = 16
TR, TCW = 8, 128

_NC, _NS = 2, 16
_NW = _NC * _NS
_BPW = B // _NW


def _sc_gather_select(P_vocab, attn_dist, pg, tidx, cidx):
    mesh = plsc.VectorSubcoreMesh(core_axis_name="c", subcore_axis_name="s")

    @functools.partial(
        pl.kernel,
        mesh=mesh,
        compiler_params=pltpu.CompilerParams(needs_layout_passes=False),
        out_type=jax.ShapeDtypeStruct((B,), jnp.float32),
        scratch_types=[
            pltpu.VMEM((_BPW,), jnp.int32),
            pltpu.VMEM((_BPW,), jnp.int32),
            pltpu.VMEM((_BPW,), jnp.float32),
            pltpu.VMEM((_BPW, TR, TCW), jnp.float32),
            pltpu.VMEM((_BPW, S), jnp.float32),
            pltpu.VMEM((_BPW,), jnp.float32),
            pltpu.SemaphoreType.DMA,
            pltpu.SemaphoreType.DMA,
        ],
    )
    def k(pv_hbm, ad_hbm, pg_hbm, tidx_hbm, cidx_hbm, out_hbm,
          tidx_v, cidx_v, pg_v, win_v, att_v, a_v,
          sem_w, sem_a):
        wid = lax.axis_index("s") * _NC + lax.axis_index("c")
        base = wid * _BPW

        pltpu.sync_copy(tidx_hbm.at[pl.ds(base, _BPW)], tidx_v)
        pltpu.sync_copy(cidx_hbm.at[pl.ds(base, _BPW)], cidx_v)
        pltpu.sync_copy(pg_hbm.at[pl.ds(base, _BPW)], pg_v)

        cp_att = pltpu.async_copy(ad_hbm.at[pl.ds(base, _BPW), :], att_v,
                                  sem_a)

        copies = []
        for kk in range(_BPW // L):
            tch = tidx_v[pl.ds(kk * L, L)]
            tch = jnp.minimum(jnp.maximum(tch, 0), V - 1)
            for jj in range(L):
                j = kk * L + jj
                col0 = pl.multiple_of((tch[jj] >> 7) << 7, TCW)
                row0 = pl.multiple_of(base + (j & ~(TR - 1)), TR)
                copies.append(pltpu.async_copy(
                    pv_hbm.at[pl.ds(row0, TR), pl.ds(col0, TCW)],
                    win_v.at[j], sem_w))
        cp_att.wait()
        for cp in copies:
            cp.wait()

        for kk in range(_BPW // L):
            sl = pl.ds(kk * L, L)
            rows = kk * L + lax.iota(jnp.int32, L)
            tv = tidx_v[sl]
            tv_safe = jnp.minimum(jnp.maximum(tv, 0), V - 1)
            cv = cidx_v[sl]
            pgv = pg_v[sl]
            pv16 = plsc.load_gather(
                win_v, [rows, rows & (TR - 1), tv_safe & (TCW - 1)])
            pc16 = plsc.load_gather(att_v, [rows, cv])
            gen = pgv * pv16 + EPS
            cpy = (1.0 - pgv) * pc16 + EPS
            a_v[sl] = jnp.where(tv < V, gen, cpy)

        pltpu.sync_copy(a_v, out_hbm.at[pl.ds(base, _BPW)])

    return k(P_vocab, attn_dist, pg, tidx, cidx)


def _tc_neg_mean_log(a):
    def body(a_ref, o_ref):
        x = a_ref[...]
        o_ref[0, 0] = -jnp.sum(jnp.log(x)) * (1.0 / B)

    out = pl.pallas_call(
        body,
        out_shape=jax.ShapeDtypeStruct((1, 1), jnp.float32),
        in_specs=[pl.BlockSpec(memory_space=pltpu.VMEM)],
        out_specs=pl.BlockSpec(memory_space=pltpu.SMEM),
    )(a.reshape(8, B // 8))
    return out[0, 0]


def kernel(P_vocab, attn_dist, p_gen, target_idx, copy_idx):
    pg = p_gen.reshape(B)
    tidx = target_idx.astype(jnp.int32)
    cidx = copy_idx.astype(jnp.int32)
    a = _sc_gather_select(P_vocab, attn_dist, pg, tidx, cidx)
    return _tc_neg_mean_log(a)


# speedup vs baseline: 31.9259x; 13.6568x over previous
import functools

import jax
import jax.numpy as jnp
from jax import lax
from jax.experimental import pallas as pl
from jax.experimental.pallas import tpu as pltpu
from jax.experimental.pallas import tpu_sc as plsc

EPS = 1e-12
B = 1024
V = 100000
S = 200
L = 16
TR, TCW = 8, 128

_NC, _NS = 2, 16
_NW = _NC * _NS
_BPW = B // _NW


def _sc_gather_select(pv_t, ad_t, pg, tidx, cidx):
    mesh = plsc.VectorSubcoreMesh(core_axis_name="c", subcore_axis_name="s")

    @functools.partial(
        pl.kernel,
        mesh=mesh,
        compiler_params=pltpu.CompilerParams(needs_layout_passes=False),
        out_type=jax.ShapeDtypeStruct((B,), jnp.float32),
        scratch_types=[
            pltpu.VMEM((_BPW,), jnp.int32),
            pltpu.VMEM((_BPW,), jnp.int32),
            pltpu.VMEM((_BPW,), jnp.float32),
            pltpu.VMEM((_BPW, TR, TCW), jnp.float32),
            pltpu.VMEM((_BPW, TR, TCW), jnp.float32),
            pltpu.VMEM((_BPW,), jnp.float32),
            pltpu.SemaphoreType.DMA,
        ],
    )
    def k(pv_hbm, ad_hbm, pg_hbm, tidx_hbm, cidx_hbm, out_hbm,
          tidx_v, cidx_v, pg_v, win_v, att_v, a_v, sem):
        wid = lax.axis_index("s") * _NC + lax.axis_index("c")
        base = wid * _BPW
        colt = pl.multiple_of((base >> 7) << 7, TCW)
        colbase = base & (TCW - 1)

        pltpu.sync_copy(tidx_hbm.at[pl.ds(base, _BPW)], tidx_v)
        pltpu.sync_copy(cidx_hbm.at[pl.ds(base, _BPW)], cidx_v)
        pltpu.sync_copy(pg_hbm.at[pl.ds(base, _BPW)], pg_v)

        copies = []
        for kk in range(_BPW // L):
            tch = tidx_v[pl.ds(kk * L, L)]
            tch = jnp.minimum(jnp.maximum(tch, 0), V - 1)
            cch = cidx_v[pl.ds(kk * L, L)]
            for jj in range(L):
                j = kk * L + jj
                trow = pl.multiple_of((tch[jj] >> 3) << 3, TR)
                copies.append(pltpu.async_copy(
                    pv_hbm.at[pl.ds(trow, TR), pl.ds(colt, TCW)],
                    win_v.at[j], sem))
                crow = pl.multiple_of((cch[jj] >> 3) << 3, TR)
                copies.append(pltpu.async_copy(
                    ad_hbm.at[pl.ds(crow, TR), pl.ds(colt, TCW)],
                    att_v.at[j], sem))
        for cp in copies:
            cp.wait()

        for kk in range(_BPW // L):
            sl = pl.ds(kk * L, L)
            rows = kk * L + lax.iota(jnp.int32, L)
            cols = colbase + rows
            tv = tidx_v[sl]
            tv_safe = jnp.minimum(jnp.maximum(tv, 0), V - 1)
            cv = cidx_v[sl]
            pgv = pg_v[sl]
            pv16 = plsc.load_gather(win_v, [rows, tv_safe & (TR - 1), cols])
            pc16 = plsc.load_gather(att_v, [rows, cv & (TR - 1), cols])
            gen = pgv * pv16 + EPS
            cpy = (1.0 - pgv) * pc16 + EPS
            a_v[sl] = jnp.where(tv < V, gen, cpy)

        pltpu.sync_copy(a_v, out_hbm.at[pl.ds(base, _BPW)])

    return k(pv_t, ad_t, pg, tidx, cidx)


def _tc_neg_mean_log(a):
    def body(a_ref, o_ref):
        x = a_ref[...]
        o_ref[0, 0] = -jnp.sum(jnp.log(x)) * (1.0 / B)

    out = pl.pallas_call(
        body,
        out_shape=jax.ShapeDtypeStruct((1, 1), jnp.float32),
        in_specs=[pl.BlockSpec(memory_space=pltpu.VMEM)],
        out_specs=pl.BlockSpec(memory_space=pltpu.SMEM),
    )(a.reshape(8, B // 8))
    return out[0, 0]


def kernel(P_vocab, attn_dist, p_gen, target_idx, copy_idx):
    pg = p_gen.reshape(B)
    tidx = target_idx.astype(jnp.int32)
    cidx = copy_idx.astype(jnp.int32)
    a = _sc_gather_select(P_vocab.T, attn_dist.T, pg, tidx, cidx)
    return _tc_neg_mean_log(a)


# speedup vs baseline: 37.3810x; 1.1709x over previous
import functools

import jax
import jax.numpy as jnp
from jax import lax
from jax.experimental import pallas as pl
from jax.experimental.pallas import tpu as pltpu
from jax.experimental.pallas import tpu_sc as plsc

EPS = 1e-12
B = 1024
V = 100000
S = 200
L = 16
TR, TCW = 8, 128

_NC, _NS = 2, 16
_NW = _NC * _NS
_BPW = B // _NW


def _sc_gather_select(pv_t, ad_t, pg, tidx, cidx):
    mesh = plsc.VectorSubcoreMesh(core_axis_name="c", subcore_axis_name="s")

    @functools.partial(
        pl.kernel,
        mesh=mesh,
        compiler_params=pltpu.CompilerParams(needs_layout_passes=False),
        out_type=jax.ShapeDtypeStruct((B,), jnp.float32),
        scratch_types=[
            pltpu.VMEM((_BPW,), jnp.int32),
            pltpu.VMEM((_BPW,), jnp.int32),
            pltpu.VMEM((_BPW,), jnp.float32),
            pltpu.VMEM((_BPW, TCW), jnp.float32),
            pltpu.VMEM((_BPW, TCW), jnp.float32),
            pltpu.VMEM((_BPW,), jnp.float32),
            pltpu.SemaphoreType.DMA,
        ],
    )
    def k(pv_hbm, ad_hbm, pg_hbm, tidx_hbm, cidx_hbm, out_hbm,
          tidx_v, cidx_v, pg_v, win_v, att_v, a_v, sem):
        wid = lax.axis_index("s") * _NC + lax.axis_index("c")
        base = wid * _BPW
        colt = pl.multiple_of((base >> 7) << 7, TCW)
        colbase = base & (TCW - 1)

        pltpu.sync_copy(tidx_hbm.at[pl.ds(base, _BPW)], tidx_v)
        pltpu.sync_copy(cidx_hbm.at[pl.ds(base, _BPW)], cidx_v)
        pltpu.sync_copy(pg_hbm.at[pl.ds(base, _BPW)], pg_v)

        cp0 = pltpu.async_copy(
            pv_hbm.at[tidx_v, pl.ds(colt, TCW)], win_v, sem)
        cp1 = pltpu.async_copy(
            ad_hbm.at[cidx_v, pl.ds(colt, TCW)], att_v, sem)
        cp0.wait()
        cp1.wait()

        for kk in range(_BPW // L):
            sl = pl.ds(kk * L, L)
            rows = kk * L + lax.iota(jnp.int32, L)
            cols = colbase + rows
            tv = tidx_v[sl]
            cv = cidx_v[sl]
            pgv = pg_v[sl]
            pv16 = plsc.load_gather(win_v, [rows, cols])
            pc16 = plsc.load_gather(att_v, [rows, cols])
            gen = pgv * pv16 + EPS
            cpy = (1.0 - pgv) * pc16 + EPS
            a_v[sl] = jnp.where(tv < V, gen, cpy)

        pltpu.sync_copy(a_v, out_hbm.at[pl.ds(base, _BPW)])

    return k(pv_t, ad_t, pg, tidx, cidx)


def _tc_neg_mean_log(a):
    def body(a_ref, o_ref):
        x = a_ref[...]
        o_ref[0, 0] = -jnp.sum(jnp.log(x)) * (1.0 / B)

    out = pl.pallas_call(
        body,
        out_shape=jax.ShapeDtypeStruct((1, 1), jnp.float32),
        in_specs=[pl.BlockSpec(memory_space=pltpu.VMEM)],
        out_specs=pl.BlockSpec(memory_space=pltpu.SMEM),
    )(a.reshape(8, B // 8))
    return out[0, 0]


def kernel(P_vocab, attn_dist, p_gen, target_idx, copy_idx):
    pg = p_gen.reshape(B)
    tidx = target_idx.astype(jnp.int32)
    cidx = copy_idx.astype(jnp.int32)
    a = _sc_gather_select(P_vocab.T, attn_dist.T, pg, tidx, cidx)
    return _tc_neg_mean_log(a)


# speedup vs baseline: 38.6458x; 1.0338x over previous
import functools

import jax
import jax.numpy as jnp
from jax import lax
from jax.experimental import pallas as pl
from jax.experimental.pallas import tpu as pltpu
from jax.experimental.pallas import tpu_sc as plsc

EPS = 1e-12
B = 1024
V = 100000
S = 200
L = 16
TR, TCW = 8, 128

_NC, _NS = 2, 16
_NW = _NC * _NS
_BPW = B // _NW


def _sc_gather_select(pv_t, ad_t, pg, tidx, cidx):
    mesh = plsc.VectorSubcoreMesh(core_axis_name="c", subcore_axis_name="s")

    @functools.partial(
        pl.kernel,
        mesh=mesh,
        compiler_params=pltpu.CompilerParams(needs_layout_passes=False),
        out_type=jax.ShapeDtypeStruct((B,), jnp.float32),
        scratch_types=[
            pltpu.VMEM((_BPW,), jnp.int32),
            pltpu.VMEM((_BPW,), jnp.int32),
            pltpu.VMEM((_BPW,), jnp.float32),
            pltpu.VMEM((_BPW, TCW), jnp.float32),
            pltpu.VMEM((_BPW, TCW), jnp.float32),
            pltpu.VMEM((_BPW,), jnp.float32),
            pltpu.SemaphoreType.DMA,
        ],
    )
    def k(pv_hbm, ad_hbm, pg_hbm, tidx_hbm, cidx_hbm, out_hbm,
          tidx_v, cidx_v, pg_v, win_v, att_v, a_v, sem):
        wid = lax.axis_index("s") * _NC + lax.axis_index("c")
        base = wid * _BPW
        colt = pl.multiple_of((base >> 7) << 7, TCW)
        colbase = base & (TCW - 1)

        cpt = pltpu.async_copy(tidx_hbm.at[pl.ds(base, _BPW)], tidx_v, sem)
        cpc = pltpu.async_copy(cidx_hbm.at[pl.ds(base, _BPW)], cidx_v, sem)
        cpp = pltpu.async_copy(pg_hbm.at[pl.ds(base, _BPW)], pg_v, sem)
        cpt.wait()
        cpc.wait()
        cpp.wait()

        cp0 = pltpu.async_copy(
            pv_hbm.at[tidx_v, pl.ds(colt, TCW)], win_v, sem)
        cp1 = pltpu.async_copy(
            ad_hbm.at[cidx_v, pl.ds(colt, TCW)], att_v, sem)
        cp0.wait()
        cp1.wait()

        for kk in range(_BPW // L):
            sl = pl.ds(kk * L, L)
            rows = kk * L + lax.iota(jnp.int32, L)
            cols = colbase + rows
            tv = tidx_v[sl]
            cv = cidx_v[sl]
            pgv = pg_v[sl]
            pv16 = plsc.load_gather(win_v, [rows, cols])
            pc16 = plsc.load_gather(att_v, [rows, cols])
            gen = pgv * pv16 + EPS
            cpy = (1.0 - pgv) * pc16 + EPS
            a_v[sl] = jnp.where(tv < V, gen, cpy)

        pltpu.sync_copy(a_v, out_hbm.at[pl.ds(base, _BPW)])

    return k(pv_t, ad_t, pg, tidx, cidx)


def _tc_neg_mean_log(a):
    def body(a_ref, o_ref):
        x = a_ref[...]
        o_ref[0, 0] = -jnp.sum(jnp.log(x)) * (1.0 / B)

    out = pl.pallas_call(
        body,
        out_shape=jax.ShapeDtypeStruct((1, 1), jnp.float32),
        in_specs=[pl.BlockSpec(memory_space=pltpu.VMEM)],
        out_specs=pl.BlockSpec(memory_space=pltpu.SMEM),
    )(a.reshape(8, B // 8))
    return out[0, 0]


def kernel(P_vocab, attn_dist, p_gen, target_idx, copy_idx):
    pg = p_gen.reshape(B)
    tidx = target_idx.astype(jnp.int32)
    cidx = copy_idx.astype(jnp.int32)
    a = _sc_gather_select(P_vocab.T, attn_dist.T, pg, tidx, cidx)
    return _tc_neg_mean_log(a)


# speedup vs baseline: 38.7177x; 1.0019x over previous
import functools

import jax
import jax.numpy as jnp
from jax import lax
from jax.experimental import pallas as pl
from jax.experimental.pallas import tpu as pltpu
from jax.experimental.pallas import tpu_sc as plsc

EPS = 1e-12
B = 1024
V = 100000
S = 200
L = 16
TR, TCW = 8, 128

_NC, _NS = 2, 16
_NW = _NC * _NS
_BPW = B // _NW


def _sc_gather_select(pv_t, ad_t, pg, tidx, cidx):
    mesh = plsc.VectorSubcoreMesh(core_axis_name="c", subcore_axis_name="s")

    @functools.partial(
        pl.kernel,
        mesh=mesh,
        compiler_params=pltpu.CompilerParams(
            needs_layout_passes=False,
            skip_device_barrier=True,
            disable_bounds_checks=True,
            disable_semaphore_checks=True,
        ),
        out_type=jax.ShapeDtypeStruct((B,), jnp.float32),
        scratch_types=[
            pltpu.VMEM((_BPW,), jnp.int32),
            pltpu.VMEM((_BPW,), jnp.int32),
            pltpu.VMEM((_BPW,), jnp.float32),
            pltpu.VMEM((_BPW, TCW), jnp.float32),
            pltpu.VMEM((_BPW, TCW), jnp.float32),
            pltpu.VMEM((_BPW,), jnp.float32),
            pltpu.SemaphoreType.DMA,
        ],
    )
    def k(pv_hbm, ad_hbm, pg_hbm, tidx_hbm, cidx_hbm, out_hbm,
          tidx_v, cidx_v, pg_v, win_v, att_v, a_v, sem):
        wid = lax.axis_index("s") * _NC + lax.axis_index("c")
        base = wid * _BPW
        colt = pl.multiple_of((base >> 7) << 7, TCW)
        colbase = base & (TCW - 1)

        cpt = pltpu.async_copy(tidx_hbm.at[pl.ds(base, _BPW)], tidx_v, sem)
        cpc = pltpu.async_copy(cidx_hbm.at[pl.ds(base, _BPW)], cidx_v, sem)
        cpp = pltpu.async_copy(pg_hbm.at[pl.ds(base, _BPW)], pg_v, sem)
        cpt.wait()
        cpc.wait()
        cpp.wait()

        cp0 = pltpu.async_copy(
            pv_hbm.at[tidx_v, pl.ds(colt, TCW)], win_v, sem)
        cp1 = pltpu.async_copy(
            ad_hbm.at[cidx_v, pl.ds(colt, TCW)], att_v, sem)
        cp0.wait()
        cp1.wait()

        for kk in range(_BPW // L):
            sl = pl.ds(kk * L, L)
            rows = kk * L + lax.iota(jnp.int32, L)
            cols = colbase + rows
            tv = tidx_v[sl]
            cv = cidx_v[sl]
            pgv = pg_v[sl]
            pv16 = plsc.load_gather(win_v, [rows, cols])
            pc16 = plsc.load_gather(att_v, [rows, cols])
            gen = pgv * pv16 + EPS
            cpy = (1.0 - pgv) * pc16 + EPS
            a_v[sl] = jnp.where(tv < V, gen, cpy)

        pltpu.sync_copy(a_v, out_hbm.at[pl.ds(base, _BPW)])

    return k(pv_t, ad_t, pg, tidx, cidx)


def _tc_neg_mean_log(a):
    def body(a_ref, o_ref):
        x = a_ref[...]
        o_ref[0, 0] = -jnp.sum(jnp.log(x)) * (1.0 / B)

    out = pl.pallas_call(
        body,
        out_shape=jax.ShapeDtypeStruct((1, 1), jnp.float32),
        in_specs=[pl.BlockSpec(memory_space=pltpu.VMEM)],
        out_specs=pl.BlockSpec(memory_space=pltpu.SMEM),
    )(a.reshape(8, B // 8))
    return out[0, 0]


def kernel(P_vocab, attn_dist, p_gen, target_idx, copy_idx):
    pg = p_gen.reshape(B)
    tidx = target_idx.astype(jnp.int32)
    cidx = copy_idx.astype(jnp.int32)
    a = _sc_gather_select(P_vocab.T, attn_dist.T, pg, tidx, cidx)
    return _tc_neg_mean_log(a)
